# baseline (speedup 1.0000x reference)
import functools

import jax
import jax.numpy as jnp
from jax import lax
from jax.experimental import pallas as pl
from jax.experimental.pallas import tpu as pltpu
from jax.experimental.pallas import tpu_sc as plsc

_VOCAB = 150528
_D = 1024
_B = 4
_S = 2048
_MASK_TOKEN = 150001
_BOS_TOKEN = 150004

_NC, _NS = 2, 16
_NW = _NC * _NS
_BS = _B * _S
_RPW = _BS // _NW
_CH = 32
_NCHUNK = _RPW // _CH


_SPW = _S // _NW
_CHS = _CH // _B


def _sc_gather_body(table_hbm, idx_hbm, out_hbm, idx_v, rows_v,
                    gsem0, gsem1, ssem0, ssem1):
    wid = lax.axis_index("s") * _NC + lax.axis_index("c")
    base = wid * _RPW
    s_base = wid * _SPW
    pltpu.sync_copy(idx_hbm.at[pl.ds(base, _RPW)], idx_v)
    gsems = (gsem0, gsem1)
    ssems = (ssem0, ssem1)
    gat = [None, None]
    sca = [[], []]
    gat[0] = pltpu.async_copy(
        table_hbm.at[idx_v.at[pl.ds(0, _CH)]], rows_v.at[0], gsems[0])
    for c in range(_NCHUNK):
        buf = c % 2
        nxt = c + 1
        if nxt < _NCHUNK:
            nbuf = nxt % 2
            for h in sca[nbuf]:
                h.wait()
            sca[nbuf] = []
            gat[nbuf] = pltpu.async_copy(
                table_hbm.at[idx_v.at[pl.ds(nxt * _CH, _CH)]],
                rows_v.at[nbuf], gsems[nbuf])
        gat[buf].wait()
        for k in range(_CHS):
            sca[buf].append(pltpu.async_copy(
                rows_v.at[buf, pl.ds(k * _B, _B)],
                out_hbm.at[s_base + c * _CHS + k], ssems[buf]))
    for lst in sca:
        for h in lst:
            h.wait()


@functools.cache
def _sc_gather():
    return pl.kernel(
        _sc_gather_body,
        mesh=plsc.VectorSubcoreMesh(core_axis_name="c", subcore_axis_name="s"),
        out_type=jax.ShapeDtypeStruct((_S, _B, _D), jnp.float32),
        scratch_types=[
            pltpu.VMEM((_RPW,), jnp.int32),
            pltpu.VMEM((2, _CH, _D), jnp.float32),
            pltpu.SemaphoreType.DMA,
            pltpu.SemaphoreType.DMA,
            pltpu.SemaphoreType.DMA,
            pltpu.SemaphoreType.DMA,
        ],
    )


_TI = 512


def _mask_body(ids_ref, pos_ref, mask_ref):
    it = pl.program_id(1)
    ids2 = ids_ref[0]
    ar2 = lax.broadcasted_iota(jnp.int32, (1, _S), 1)
    ctx = jnp.min(jnp.where(ids2 == _BOS_TOKEN, ar2, _S))
    mpos = jnp.min(jnp.where(ids2 == _MASK_TOKEN, ar2, _S))
    i0 = it * _TI
    row1 = lax.broadcasted_iota(jnp.int32, (_TI, _S), 0) + (i0 + 1)
    t = jnp.maximum(row1, ctx)
    col = lax.broadcasted_iota(jnp.int32, (_TI, _S), 1)
    mask_ref[0, 0, :, :] = (col >= t).astype(jnp.int8)
    pos = jnp.where(ar2 >= ctx, mpos, ar2)
    bpos = jnp.where(ar2 < ctx, 0, ar2 - ctx + 1)
    pos_ref[0, :, :] = jnp.concatenate([pos, bpos], axis=0)


def _mask_call(input_ids):
    return pl.pallas_call(
        _mask_body,
        grid=(_B, _S // _TI),
        in_specs=[pl.BlockSpec((1, 1, _S), lambda b, i: (b, 0, 0))],
        out_specs=[
            pl.BlockSpec((1, 2, _S), lambda b, i: (b, 0, 0)),
            pl.BlockSpec((1, 1, _TI, _S), lambda b, i: (b, 0, i, 0)),
        ],
        out_shape=[
            jax.ShapeDtypeStruct((_B, 2, _S), jnp.int32),
            jax.ShapeDtypeStruct((_B, 1, _S, _S), jnp.int8),
        ],
    )(input_ids.reshape(_B, 1, _S))


def kernel(input_ids, labels, weight):
    ids_flat = input_ids.T.reshape(-1)
    hidden_states = _sc_gather()(weight, ids_flat)
    position_ids, mask_i8 = _mask_call(input_ids)
    attention_mask = mask_i8.astype(jnp.bool_)
    return hidden_states, position_ids, attention_mask, labels


# speedup vs baseline: 1.0121x; 1.0121x over previous
import functools

import jax
import jax.numpy as jnp
from jax import lax
from jax.experimental import pallas as pl
from jax.experimental.pallas import tpu as pltpu
from jax.experimental.pallas import tpu_sc as plsc

_VOCAB = 150528
_D = 1024
_B = 4
_S = 2048
_MASK_TOKEN = 150001
_BOS_TOKEN = 150004

_NC, _NS = 2, 16
_NW = _NC * _NS
_BS = _B * _S
_RPW = _BS // _NW
_CH = 32
_NCHUNK = _RPW // _CH


_SPW = _S // _NW
_CHS = _CH // _B


_NGRP = _CH // 16


def _sc_gather_body(table_hbm, idx_hbm, out_hbm, idx_v, rows_v, sem0, sem1):
    wid = lax.axis_index("s") * _NC + lax.axis_index("c")
    base = wid * _RPW
    s_base = wid * _SPW
    pltpu.sync_copy(idx_hbm.at[pl.ds(base, _RPW)], idx_v)
    sems = (sem0, sem1)
    gat = [None, None]
    gat[0] = pltpu.async_copy(
        table_hbm.at[idx_v.at[pl.ds(0, _CH)]], rows_v.at[0], sems[0])
    for c in range(_NCHUNK):
        buf = c % 2
        nxt = c + 1
        if nxt < _NCHUNK:
            gat[nxt % 2] = pltpu.async_copy(
                table_hbm.at[idx_v.at[pl.ds(nxt * _CH, _CH)]],
                rows_v.at[nxt % 2], sems[nxt % 2])
        gat[buf].wait()
        for k in range(_CHS):
            pltpu.sync_copy(rows_v.at[buf, pl.ds(k * _B, _B)],
                            out_hbm.at[s_base + c * _CHS + k])


@functools.cache
def _sc_gather():
    return pl.kernel(
        _sc_gather_body,
        mesh=plsc.VectorSubcoreMesh(core_axis_name="c", subcore_axis_name="s"),
        out_type=jax.ShapeDtypeStruct((_S, _B, _D), jnp.float32),
        scratch_types=[
            pltpu.VMEM((_RPW,), jnp.int32),
            pltpu.VMEM((2, _CH, _D), jnp.float32),
            pltpu.SemaphoreType.DMA,
            pltpu.SemaphoreType.DMA,
        ],
    )


_TI = 512


def _mask_body(ids_ref, pos_ref, mask_ref):
    it = pl.program_id(1)
    ids2 = ids_ref[0]
    ar2 = lax.broadcasted_iota(jnp.int32, (1, _S), 1)
    ctx = jnp.min(jnp.where(ids2 == _BOS_TOKEN, ar2, _S))
    mpos = jnp.min(jnp.where(ids2 == _MASK_TOKEN, ar2, _S))
    i0 = it * _TI
    row1 = lax.broadcasted_iota(jnp.int32, (_TI, 1), 0) + (i0 + 1)
    t16 = jnp.maximum(row1, ctx).astype(jnp.int16)
    col = lax.broadcasted_iota(jnp.int16, (_TI, _S), 1)
    mask_ref[0, 0, :, :] = (col >= t16).astype(jnp.int8)
    pos = jnp.where(ar2 >= ctx, mpos, ar2)
    bpos = jnp.where(ar2 < ctx, 0, ar2 - ctx + 1)
    pos_ref[0, :, :] = jnp.concatenate([pos, bpos], axis=0)


def _mask_call(input_ids):
    return pl.pallas_call(
        _mask_body,
        grid=(_B, _S // _TI),
        in_specs=[pl.BlockSpec((1, 1, _S), lambda b, i: (b, 0, 0))],
        out_specs=[
            pl.BlockSpec((1, 2, _S), lambda b, i: (b, 0, 0)),
            pl.BlockSpec((1, 1, _TI, _S), lambda b, i: (b, 0, i, 0)),
        ],
        out_shape=[
            jax.ShapeDtypeStruct((_B, 2, _S), jnp.int32),
            jax.ShapeDtypeStruct((_B, 1, _S, _S), jnp.int8),
        ],
    )(input_ids.reshape(_B, 1, _S))


def kernel(input_ids, labels, weight):
    ids_flat = input_ids.T.reshape(-1)
    hidden_states = _sc_gather()(weight, ids_flat)
    position_ids, mask_i8 = _mask_call(input_ids)
    attention_mask = mask_i8.astype(jnp.bool_)
    return hidden_states, position_ids, attention_mask, labels


# speedup vs baseline: 1.0198x; 1.0077x over previous
import functools

import jax
import jax.numpy as jnp
from jax import lax
from jax.experimental import pallas as pl
from jax.experimental.pallas import tpu as pltpu
from jax.experimental.pallas import tpu_sc as plsc

_VOCAB = 150528
_D = 1024
_B = 4
_S = 2048
_MASK_TOKEN = 150001
_BOS_TOKEN = 150004

_NC, _NS = 2, 16
_NW = _NC * _NS
_BS = _B * _S
_RPW = _BS // _NW
_CH = 16
_NBUF = 4
_NCHUNK = _RPW // _CH


_SPW = _S // _NW
_CHS = _CH // _B


_NGRP = _CH // 16


def _sc_gather_body(table_hbm, idx_hbm, out_hbm, idx_v, rows_v,
                    sem0, sem1, sem2, sem3):
    wid = lax.axis_index("s") * _NC + lax.axis_index("c")
    base = wid * _RPW
    s_base = wid * _SPW
    pltpu.sync_copy(idx_hbm.at[pl.ds(base, _RPW)], idx_v)
    sems = (sem0, sem1, sem2, sem3)
    gat = [None] * _NBUF

    def start(c):
        gat[c % _NBUF] = pltpu.async_copy(
            table_hbm.at[idx_v.at[pl.ds(c * _CH, _CH)]],
            rows_v.at[c % _NBUF], sems[c % _NBUF])

    for c in range(_NBUF - 1):
        start(c)
    for c in range(_NCHUNK):
        nxt = c + _NBUF - 1
        if nxt < _NCHUNK:
            start(nxt)
        gat[c % _NBUF].wait()
        for k in range(_CHS):
            pltpu.sync_copy(rows_v.at[c % _NBUF, pl.ds(k * _B, _B)],
                            out_hbm.at[s_base + c * _CHS + k])


@functools.cache
def _sc_gather():
    return pl.kernel(
        _sc_gather_body,
        mesh=plsc.VectorSubcoreMesh(core_axis_name="c", subcore_axis_name="s"),
        out_type=jax.ShapeDtypeStruct((_S, _B, _D), jnp.float32),
        scratch_types=[
            pltpu.VMEM((_RPW,), jnp.int32),
            pltpu.VMEM((_NBUF, _CH, _D), jnp.float32),
            pltpu.SemaphoreType.DMA,
            pltpu.SemaphoreType.DMA,
            pltpu.SemaphoreType.DMA,
            pltpu.SemaphoreType.DMA,
        ],
    )


_TI = 512


def _mask_body(ids_ref, pos_ref, mask_ref):
    it = pl.program_id(1)
    ids2 = ids_ref[0]
    ar2 = lax.broadcasted_iota(jnp.int32, (1, _S), 1)
    ctx = jnp.min(jnp.where(ids2 == _BOS_TOKEN, ar2, _S))
    mpos = jnp.min(jnp.where(ids2 == _MASK_TOKEN, ar2, _S))
    i0 = it * _TI
    row1 = lax.broadcasted_iota(jnp.int32, (_TI, 1), 0) + (i0 + 1)
    t16 = jnp.maximum(row1, ctx).astype(jnp.int16)
    col = lax.broadcasted_iota(jnp.int16, (_TI, _S), 1)
    mask_ref[0, 0, :, :] = (col >= t16).astype(jnp.int8)
    pos = jnp.where(ar2 >= ctx, mpos, ar2)
    bpos = jnp.where(ar2 < ctx, 0, ar2 - ctx + 1)
    pos_ref[0, :, :] = jnp.concatenate([pos, bpos], axis=0)


def _mask_call(input_ids):
    return pl.pallas_call(
        _mask_body,
        grid=(_B, _S // _TI),
        in_specs=[pl.BlockSpec((1, 1, _S), lambda b, i: (b, 0, 0))],
        out_specs=[
            pl.BlockSpec((1, 2, _S), lambda b, i: (b, 0, 0)),
            pl.BlockSpec((1, 1, _TI, _S), lambda b, i: (b, 0, i, 0)),
        ],
        out_shape=[
            jax.ShapeDtypeStruct((_B, 2, _S), jnp.int32),
            jax.ShapeDtypeStruct((_B, 1, _S, _S), jnp.int8),
        ],
    )(input_ids.reshape(_B, 1, _S))


def kernel(input_ids, labels, weight):
    ids_flat = input_ids.T.reshape(-1)
    hidden_states = _sc_gather()(weight, ids_flat)
    position_ids, mask_i8 = _mask_call(input_ids)
    attention_mask = mask_i8.astype(jnp.bool_)
    return hidden_states, position_ids, attention_mask, labels


# speedup vs baseline: 1.0309x; 1.0108x over previous
import functools

import jax
import jax.numpy as jnp
from jax import lax
from jax.experimental import pallas as pl
from jax.experimental.pallas import tpu as pltpu
from jax.experimental.pallas import tpu_sc as plsc

_VOCAB = 150528
_D = 1024
_B = 4
_S = 2048
_MASK_TOKEN = 150001
_BOS_TOKEN = 150004

_NC, _NS = 2, 16
_NW = _NC * _NS
_BS = _B * _S
_RPW = _BS // _NW
_CH = 16
_NBUF = 6
_NCHUNK = _RPW // _CH


_SPW = _S // _NW
_CHS = _CH // _B


_NGRP = _CH // 16


def _sc_gather_body(table_hbm, idx_hbm, out_hbm, idx_v, rows_v,
                    sem0, sem1, sem2, sem3, sem4, sem5):
    wid = lax.axis_index("s") * _NC + lax.axis_index("c")
    base = wid * _RPW
    s_base = wid * _SPW
    pltpu.sync_copy(idx_hbm.at[pl.ds(base, _RPW)], idx_v)
    sems = (sem0, sem1, sem2, sem3, sem4, sem5)
    gat = [None] * _NBUF

    def start(c):
        gat[c % _NBUF] = pltpu.async_copy(
            table_hbm.at[idx_v.at[pl.ds(c * _CH, _CH)]],
            rows_v.at[c % _NBUF], sems[c % _NBUF])

    for c in range(_NBUF - 1):
        start(c)
    for c in range(_NCHUNK):
        nxt = c + _NBUF - 1
        if nxt < _NCHUNK:
            start(nxt)
        gat[c % _NBUF].wait()
        for k in range(_CHS):
            pltpu.sync_copy(rows_v.at[c % _NBUF, pl.ds(k * _B, _B)],
                            out_hbm.at[s_base + c * _CHS + k])


@functools.cache
def _sc_gather():
    return pl.kernel(
        _sc_gather_body,
        mesh=plsc.VectorSubcoreMesh(core_axis_name="c", subcore_axis_name="s"),
        out_type=jax.ShapeDtypeStruct((_S, _B, _D), jnp.float32),
        scratch_types=[
            pltpu.VMEM((_RPW,), jnp.int32),
            pltpu.VMEM((_NBUF, _CH, _D), jnp.float32),
            pltpu.SemaphoreType.DMA,
            pltpu.SemaphoreType.DMA,
            pltpu.SemaphoreType.DMA,
            pltpu.SemaphoreType.DMA,
            pltpu.SemaphoreType.DMA,
            pltpu.SemaphoreType.DMA,
        ],
    )


_TI = 512


def _mask_body(ids_ref, pos_ref, mask_ref):
    it = pl.program_id(1)
    ids2 = ids_ref[0]
    ar2 = lax.broadcasted_iota(jnp.int32, (1, _S), 1)
    ctx = jnp.min(jnp.where(ids2 == _BOS_TOKEN, ar2, _S))
    mpos = jnp.min(jnp.where(ids2 == _MASK_TOKEN, ar2, _S))
    i0 = it * _TI
    row1 = lax.broadcasted_iota(jnp.int32, (_TI, 1), 0) + (i0 + 1)
    t16 = jnp.maximum(row1, ctx).astype(jnp.int16)
    col = lax.broadcasted_iota(jnp.int16, (_TI, _S), 1)
    mask_ref[0, 0, :, :] = (col >= t16).astype(jnp.int8)
    pos = jnp.where(ar2 >= ctx, mpos, ar2)
    bpos = jnp.where(ar2 < ctx, 0, ar2 - ctx + 1)
    pos_ref[0, :, :] = jnp.concatenate([pos, bpos], axis=0)


def _mask_call(input_ids):
    return pl.pallas_call(
        _mask_body,
        grid=(_B, _S // _TI),
        in_specs=[pl.BlockSpec((1, 1, _S), lambda b, i: (b, 0, 0))],
        out_specs=[
            pl.BlockSpec((1, 2, _S), lambda b, i: (b, 0, 0)),
            pl.BlockSpec((1, 1, _TI, _S), lambda b, i: (b, 0, i, 0)),
        ],
        out_shape=[
            jax.ShapeDtypeStruct((_B, 2, _S), jnp.int32),
            jax.ShapeDtypeStruct((_B, 1, _S, _S), jnp.int8),
        ],
    )(input_ids.reshape(_B, 1, _S))


def kernel(input_ids, labels, weight):
    ids_flat = input_ids.T.reshape(-1)
    hidden_states = _sc_gather()(weight, ids_flat)
    position_ids, mask_i8 = _mask_call(input_ids)
    attention_mask = mask_i8.astype(jnp.bool_)
    return hidden_states, position_ids, attention_mask, labels


# speedup vs baseline: 1.0316x; 1.0007x over previous
import functools

import jax
import jax.numpy as jnp
from jax import lax
from jax.experimental import pallas as pl
from jax.experimental.pallas import tpu as pltpu
from jax.experimental.pallas import tpu_sc as plsc

_VOCAB = 150528
_D = 1024
_B = 4
_S = 2048
_MASK_TOKEN = 150001
_BOS_TOKEN = 150004

_NC, _NS = 2, 16
_NW = _NC * _NS
_BS = _B * _S
_RPW = _BS // _NW
_CH = 16
_NBUF = 6
_NCHUNK = _RPW // _CH


_SPW = _S // _NW
_CHS = _CH // _B


def _sc_gather_body(table_hbm, idx_hbm, out_hbm, idx_v, rows_v,
                    sem0, sem1, sem2, sem3, sem4, sem5):
    wid = lax.axis_index("s") * _NC + lax.axis_index("c")
    base = wid * _RPW
    s_base = wid * _SPW
    pltpu.sync_copy(idx_hbm.at[pl.ds(base, _RPW)], idx_v)
    sems = (sem0, sem1, sem2, sem3, sem4, sem5)
    gat = [None] * _NBUF

    def start(c):
        gat[c % _NBUF] = pltpu.async_copy(
            table_hbm.at[idx_v.at[pl.ds(c * _CH, _CH)]],
            rows_v.at[c % _NBUF], sems[c % _NBUF])

    for c in range(_NBUF - 1):
        start(c)
    for c in range(_NCHUNK):
        nxt = c + _NBUF - 1
        if nxt < _NCHUNK:
            start(nxt)
        gat[c % _NBUF].wait()
        for k in range(_CHS):
            pltpu.sync_copy(rows_v.at[c % _NBUF, pl.ds(k * _B, _B)],
                            out_hbm.at[s_base + c * _CHS + k])


@functools.cache
def _sc_gather():
    return pl.kernel(
        _sc_gather_body,
        mesh=plsc.VectorSubcoreMesh(core_axis_name="c", subcore_axis_name="s"),
        out_type=jax.ShapeDtypeStruct((_S, _B, _D), jnp.float32),
        scratch_types=[
            pltpu.VMEM((_RPW,), jnp.int32),
            pltpu.VMEM((_NBUF, _CH, _D), jnp.float32),
            pltpu.SemaphoreType.DMA,
            pltpu.SemaphoreType.DMA,
            pltpu.SemaphoreType.DMA,
            pltpu.SemaphoreType.DMA,
            pltpu.SemaphoreType.DMA,
            pltpu.SemaphoreType.DMA,
        ],
    )


_TI = 512


def _mask_body(ids_ref, pos_ref, mask_ref):
    it = pl.program_id(1)
    ids2 = ids_ref[0]
    ar2 = lax.broadcasted_iota(jnp.int32, (1, _S), 1)
    ctx = jnp.min(jnp.where(ids2 == _BOS_TOKEN, ar2, _S))
    mpos = jnp.min(jnp.where(ids2 == _MASK_TOKEN, ar2, _S))
    i0 = it * _TI
    row1 = lax.broadcasted_iota(jnp.int32, (_TI, 1), 0) + (i0 + 1)
    t16 = jnp.maximum(row1, ctx).astype(jnp.int16)
    col = lax.broadcasted_iota(jnp.int16, (_TI, _S), 1)
    mask_ref[0, 0, :, :] = (col >= t16).astype(jnp.int8)
    pos = jnp.where(ar2 >= ctx, mpos, ar2)
    bpos = jnp.where(ar2 < ctx, 0, ar2 - ctx + 1)
    pos_ref[0, :, :] = jnp.concatenate([pos, bpos], axis=0)


def _mask_call(input_ids):
    return pl.pallas_call(
        _mask_body,
        grid=(_B, _S // _TI),
        in_specs=[pl.BlockSpec((1, 1, _S), lambda b, i: (b, 0, 0))],
        out_specs=[
            pl.BlockSpec((1, 2, _S), lambda b, i: (b, 0, 0)),
            pl.BlockSpec((1, 1, _TI, _S), lambda b, i: (b, 0, i, 0)),
        ],
        out_shape=[
            jax.ShapeDtypeStruct((_B, 2, _S), jnp.int32),
            jax.ShapeDtypeStruct((_B, 1, _S, _S), jnp.int8),
        ],
    )(input_ids.reshape(_B, 1, _S))


def kernel(input_ids, labels, weight):
    ids_flat = input_ids.T.reshape(-1)
    hidden_states = _sc_gather()(weight, ids_flat)
    position_ids, mask_i8 = _mask_call(input_ids)
    attention_mask = mask_i8.astype(jnp.bool_)
    return hidden_states, position_ids, attention_mask, labels
